# Initial kernel scaffold; baseline (speedup 1.0000x reference)
import functools

import jax
import jax.numpy as jnp
from jax import lax
from jax.experimental import pallas as pl
from jax.experimental.pallas import tpu as pltpu
from jax.experimental.pallas import tpu_sc as plsc

N_NODES = 10000
N_EDGES = 320000
D = 128
N_GRAPHS = 64

NC = 2
NS = 16
N_TILES = NC * NS
EDGES_PER_TILE = N_EDGES // N_TILES
CHUNK = 80
N_CHUNKS = EDGES_PER_TILE // CHUNK
EDGE_ROWS = N_EDGES // CHUNK
ROWS_PER_TILE = N_NODES // NS


def _pre_body(x_ref, wr1_ref, wt1_ref, br1_ref, wr3_ref, wt3_ref, wlin_ref,
              br3_ref, y_ref, z_ref, u_ref):
    x = x_ref[...]
    dn = (((1,), (1,)), ((), ()))
    y_ref[...] = lax.dot_general(x, wr1_ref[...], dn,
                                 preferred_element_type=jnp.float32)
    z_ref[...] = lax.dot_general(x, wt1_ref[...], dn,
                                 preferred_element_type=jnp.float32) + br1_ref[...]
    wlin = wlin_ref[...]
    dn0 = (((1,), (0,)), ((), ()))
    u_rel = lax.dot_general(wlin, wr3_ref[...], dn0,
                            preferred_element_type=jnp.float32)
    u_root = lax.dot_general(wlin, wt3_ref[...], dn0,
                             preferred_element_type=jnp.float32)
    b_c = lax.dot_general(br3_ref[...], wlin, dn,
                          preferred_element_type=jnp.float32)
    u_ref[...] = jnp.concatenate(
        [u_rel, u_root, jnp.broadcast_to(b_c, (1, D)),
         jnp.zeros((5, D), jnp.float32)], axis=0)


def _sc_conv_body(y_hbm, src_hbm, dst_hbm, w_hbm, zero_hbm, out_hbm,
                  src_v, dst_v, w_v, rows_v, agg_sh, sem):
    cid = lax.axis_index("c")
    sid = lax.axis_index("s")
    wid = cid * NS + sid
    base = wid * N_CHUNKS
    pltpu.sync_copy(src_hbm.at[pl.ds(base, N_CHUNKS)], src_v)
    pltpu.sync_copy(dst_hbm.at[pl.ds(base, N_CHUNKS)], dst_v)
    pltpu.sync_copy(w_hbm.at[pl.ds(base, N_CHUNKS)], w_v)
    row0 = sid * ROWS_PER_TILE
    pltpu.sync_copy(zero_hbm.at[pl.ds(row0, ROWS_PER_TILE)],
                    agg_sh.at[pl.ds(row0, ROWS_PER_TILE)])
    plsc.subcore_barrier()

    def chunk_body(c, carry):
        pltpu.async_copy(y_hbm.at[src_v.at[c]], rows_v, sem).wait()
        cidx = jnp.full((16,), c, jnp.int32)

        def edge_body(e, carry2):
            wspl = plsc.load_gather(w_v, [cidx, jnp.full((16,), e, jnp.int32)])
            for u in range(D // 16):
                rows_v[e, pl.ds(u * 16, 16)] = rows_v[e, pl.ds(u * 16, 16)] * wspl
            return carry2

        lax.fori_loop(0, CHUNK, edge_body, 0)
        pltpu.sync_copy(rows_v, agg_sh.at[dst_v.at[c]], add=True)
        return carry

    lax.fori_loop(0, N_CHUNKS, chunk_body, 0)
    plsc.subcore_barrier()
    pltpu.sync_copy(agg_sh.at[pl.ds(row0, ROWS_PER_TILE)],
                    out_hbm.at[cid, pl.ds(row0, ROWS_PER_TILE)])


def _mid_body(agg_ref, z_ref, u_ref, out_ref):
    h = jnp.maximum(agg_ref[0] + agg_ref[1] + z_ref[...], 0.0)
    u2 = u_ref[0:2, :]
    out_ref[...] = lax.dot_general(h, u2, (((1,), (1,)), ((), ())),
                                   preferred_element_type=jnp.float32)


def _sc_q_body(s_hbm, src_hbm, dst_hbm, w_hbm, out_hbm,
               s_v, src_v, dst_v, w_v, q_v):
    cid = lax.axis_index("c")
    sid = lax.axis_index("s")
    wid = cid * NS + sid
    base = wid * N_CHUNKS
    pltpu.sync_copy(s_hbm, s_v)
    pltpu.sync_copy(src_hbm.at[pl.ds(base, N_CHUNKS)], src_v)
    pltpu.sync_copy(dst_hbm.at[pl.ds(base, N_CHUNKS)], dst_v)
    pltpu.sync_copy(w_hbm.at[pl.ds(base, N_CHUNKS)], w_v)

    def zero_body(i, carry):
        q_v[pl.ds(i * 16, 16)] = jnp.zeros((16,), jnp.float32)
        return carry

    lax.fori_loop(0, N_NODES // 16, zero_body, 0)

    def chunk_body(c, carry):
        for j in range(CHUNK // 16):
            s16 = src_v[c, pl.ds(j * 16, 16)]
            d16 = dst_v[c, pl.ds(j * 16, 16)]
            w16 = w_v[c, pl.ds(j * 16, 16)]
            vals = plsc.load_gather(s_v, [s16])
            plsc.addupdate_scatter(q_v, [d16], vals * w16)
        return carry

    lax.fori_loop(0, N_CHUNKS, chunk_body, 0)
    pltpu.sync_copy(q_v, out_hbm.at[wid])


def _pool_body(q_ref, sroot_ref, batch_ref, u_ref, blin_ref, out_ref):
    p = jnp.sum(q_ref[...], axis=0, keepdims=True) + sroot_ref[...] \
        + u_ref[2:3, 0:1]
    b = batch_ref[...]
    gids = lax.broadcasted_iota(jnp.int32, (N_GRAPHS, N_NODES), 0)
    oh = jnp.where(gids == b, 1.0, 0.0).astype(jnp.float32)
    dn = (((1,), (1,)), ((), ()))
    sums = lax.dot_general(oh, p, dn, preferred_element_type=jnp.float32)
    counts = lax.dot_general(oh, jnp.ones((1, N_NODES), jnp.float32), dn,
                             preferred_element_type=jnp.float32)
    res = jnp.maximum(sums / jnp.maximum(counts, 1.0) + blin_ref[...], 0.0)
    out_ref[...] = jnp.broadcast_to(res, (N_GRAPHS, D))


def kernel(x, edge_index, batch, edge_attr, W_rel1, b_rel1, W_root1,
           W_rel3, b_rel3, W_root3, W_lin, b_lin):
    f32 = jnp.float32
    src2 = edge_index[0].astype(jnp.int32).reshape(EDGE_ROWS, CHUNK)
    dst2 = edge_index[1].astype(jnp.int32).reshape(EDGE_ROWS, CHUNK)
    w2 = edge_attr.astype(f32).reshape(EDGE_ROWS, CHUNK)
    zeros_nd = jnp.zeros((N_NODES, D), f32)

    y, z, u = pl.pallas_call(
        _pre_body,
        out_shape=(
            jax.ShapeDtypeStruct((N_NODES, D), f32),
            jax.ShapeDtypeStruct((N_NODES, D), f32),
            jax.ShapeDtypeStruct((8, D), f32),
        ),
    )(x, W_rel1, W_root1, b_rel1.reshape(1, D), W_rel3, W_root3, W_lin,
      b_rel3.reshape(1, D))

    mesh = plsc.VectorSubcoreMesh(core_axis_name="c", subcore_axis_name="s")
    agg = pl.kernel(
        _sc_conv_body,
        out_type=jax.ShapeDtypeStruct((NC, N_NODES, D), f32),
        mesh=mesh,
        scratch_types=[
            pltpu.VMEM((N_CHUNKS, CHUNK), jnp.int32),
            pltpu.VMEM((N_CHUNKS, CHUNK), jnp.int32),
            pltpu.VMEM((N_CHUNKS, CHUNK), f32),
            pltpu.VMEM((CHUNK, D), f32),
            pltpu.VMEM_SHARED((N_NODES, D), f32),
            pltpu.SemaphoreType.DMA,
        ],
    )(y, src2, dst2, w2, zeros_nd)

    scat = pl.pallas_call(
        _mid_body,
        grid=(5,),
        in_specs=[
            pl.BlockSpec((NC, N_NODES // 5, D), lambda i: (0, i, 0)),
            pl.BlockSpec((N_NODES // 5, D), lambda i: (i, 0)),
            pl.BlockSpec((8, D), lambda i: (0, 0)),
        ],
        out_specs=pl.BlockSpec((N_NODES // 5, 2), lambda i: (i, 0)),
        out_shape=jax.ShapeDtypeStruct((N_NODES, 2), f32),
    )(agg, z, u)

    s_rel = scat[:, 0]
    s_root = scat[:, 1].reshape(1, N_NODES)

    q = pl.kernel(
        _sc_q_body,
        out_type=jax.ShapeDtypeStruct((N_TILES, N_NODES), f32),
        mesh=mesh,
        scratch_types=[
            pltpu.VMEM((N_NODES,), f32),
            pltpu.VMEM((N_CHUNKS, CHUNK), jnp.int32),
            pltpu.VMEM((N_CHUNKS, CHUNK), jnp.int32),
            pltpu.VMEM((N_CHUNKS, CHUNK), f32),
            pltpu.VMEM((N_NODES,), f32),
        ],
    )(s_rel, src2, dst2, w2)

    pooled = pl.pallas_call(
        _pool_body,
        out_shape=jax.ShapeDtypeStruct((N_GRAPHS, D), f32),
    )(q, s_root, batch.astype(jnp.int32).reshape(1, N_NODES), u,
      b_lin.reshape(1, 1))

    return pooled[:, :1]


# speedup vs baseline: 4.6108x; 4.6108x over previous
import jax
import jax.numpy as jnp
from jax import lax
from jax.experimental import pallas as pl
from jax.experimental.pallas import tpu as pltpu
from jax.experimental.pallas import tpu_sc as plsc

N_NODES = 10000
N_EDGES = 320000
D = 128
N_GRAPHS = 64

NC = 2
NS = 16
N_TILES = NC * NS
CHUNK = 128
CPT = 80
E_PAD = N_TILES * CPT * CHUNK
EDGE_ROWS = E_PAD // CHUNK
N_PAD = 10112
STRIPE = N_PAD // NS

def _splat_lane(v16, l):
    dn = lax.GatherDimensionNumbers(offset_dims=(), collapsed_slice_dims=(0,),
                                    start_index_map=(0,))
    idx = jnp.full((16, 1), l, jnp.int32)
    return lax.gather(v16, idx, dn, slice_sizes=(1,),
                      mode=lax.GatherScatterMode.PROMISE_IN_BOUNDS)


def _pre_body(x_ref, wr1_ref, wt1_ref, br1_ref, wr3_ref, wt3_ref, wlin_ref,
              br3_ref, y_ref, z_ref, u_ref):
    x = x_ref[...]
    dn = (((1,), (1,)), ((), ()))
    y_ref[...] = lax.dot_general(x, wr1_ref[...], dn,
                                 preferred_element_type=jnp.float32)
    z_ref[...] = lax.dot_general(x, wt1_ref[...], dn,
                                 preferred_element_type=jnp.float32) + br1_ref[...]
    wlin = wlin_ref[...]
    dn0 = (((1,), (0,)), ((), ()))
    u_rel = lax.dot_general(wlin, wr3_ref[...], dn0,
                            preferred_element_type=jnp.float32)
    u_root = lax.dot_general(wlin, wt3_ref[...], dn0,
                             preferred_element_type=jnp.float32)
    b_c = lax.dot_general(br3_ref[...], wlin, dn,
                          preferred_element_type=jnp.float32)
    u_ref[...] = jnp.concatenate(
        [u_rel, u_root, jnp.broadcast_to(b_c, (1, D)),
         jnp.zeros((5, D), jnp.float32)], axis=0)


def _fill_tile_rows(idx_v, wid):
    def body(i, carry):
        idx_v[pl.ds(i * 16, 16)] = lax.iota(jnp.int32, 16) + (wid * CPT + i * 16)
        return carry
    lax.fori_loop(0, CPT // 16, body, 0)


def _sc_conv_body(y_hbm, src_hbm, dst_hbm, w_hbm, zero_hbm, out_hbm,
                  idx_v, src_v, dst_v, w_v, rows_v, agg_sh, sem):
    cid = lax.axis_index("c")
    sid = lax.axis_index("s")
    wid = cid * NS + sid
    _fill_tile_rows(idx_v, wid)
    pltpu.async_copy(src_hbm.at[idx_v], src_v, sem).wait()
    pltpu.async_copy(dst_hbm.at[idx_v], dst_v, sem).wait()
    pltpu.async_copy(w_hbm.at[idx_v], w_v, sem).wait()
    row0 = sid * STRIPE
    pltpu.sync_copy(zero_hbm.at[pl.ds(row0, STRIPE)],
                    agg_sh.at[pl.ds(row0, STRIPE)])
    plsc.subcore_barrier()

    def chunk_body(c, carry):
        pltpu.async_copy(y_hbm.at[src_v.at[c]], rows_v, sem).wait()

        def grp_body(j, carry2):
            w16 = w_v[c, pl.ds(j * 16, 16)]
            for l in range(16):
                wspl = _splat_lane(w16, l)
                e = j * 16 + l
                for u in range(D // 16):
                    rows_v[e, pl.ds(u * 16, 16)] = \
                        rows_v[e, pl.ds(u * 16, 16)] * wspl
            return carry2

        lax.fori_loop(0, CHUNK // 16, grp_body, 0)
        pltpu.sync_copy(rows_v, agg_sh.at[dst_v.at[c]], add=True)
        return carry

    lax.fori_loop(0, CPT, chunk_body, 0)
    plsc.subcore_barrier()
    pltpu.sync_copy(agg_sh.at[pl.ds(row0, STRIPE)],
                    out_hbm.at[cid, pl.ds(row0, STRIPE)])


def _mid_body(agg_ref, z_ref, u_ref, out_ref):
    h = jnp.maximum(agg_ref[0] + agg_ref[1] + z_ref[...], 0.0)
    u2 = u_ref[0:2, :]
    out_ref[...] = lax.dot_general(h, u2, (((1,), (1,)), ((), ())),
                                   preferred_element_type=jnp.float32)


def _sc_q_body(s_hbm, src_hbm, dst_hbm, w_hbm, out_hbm,
               idx_v, s_v, src_v, dst_v, w_v, q_v, sem):
    cid = lax.axis_index("c")
    sid = lax.axis_index("s")
    wid = cid * NS + sid
    _fill_tile_rows(idx_v, wid)
    pltpu.sync_copy(s_hbm, s_v)
    pltpu.async_copy(src_hbm.at[idx_v], src_v, sem).wait()
    pltpu.async_copy(dst_hbm.at[idx_v], dst_v, sem).wait()
    pltpu.async_copy(w_hbm.at[idx_v], w_v, sem).wait()
    q1 = q_v.at[0]

    def zero_body(i, carry):
        q1[pl.ds(i * 16, 16)] = jnp.zeros((16,), jnp.float32)
        return carry

    lax.fori_loop(0, N_NODES // 16, zero_body, 0)

    def chunk_body(c, carry):
        for j in range(CHUNK // 16):
            s16 = src_v[c, pl.ds(j * 16, 16)]
            d16 = dst_v[c, pl.ds(j * 16, 16)]
            w16 = w_v[c, pl.ds(j * 16, 16)]
            vals = plsc.load_gather(s_v, [s16])
            plsc.addupdate_scatter(q1, [d16], vals * w16)
        return carry

    lax.fori_loop(0, CPT, chunk_body, 0)
    pltpu.sync_copy(q_v, out_hbm.at[wid])


def _pool_body(q_ref, sroot_ref, batch_ref, u_ref, blin_ref, out_ref):
    p = jnp.sum(q_ref[...], axis=0) + sroot_ref[...] \
        + u_ref[2:3, 0:1]
    b = batch_ref[...]
    gids = lax.broadcasted_iota(jnp.int32, (N_GRAPHS, N_NODES), 0)
    oh = jnp.where(gids == b, 1.0, 0.0).astype(jnp.float32)
    dn = (((1,), (1,)), ((), ()))
    sums = lax.dot_general(oh, p, dn, preferred_element_type=jnp.float32)
    counts = lax.dot_general(oh, jnp.ones((1, N_NODES), jnp.float32), dn,
                             preferred_element_type=jnp.float32)
    res = jnp.maximum(sums / jnp.maximum(counts, 1.0) + blin_ref[...], 0.0)
    out_ref[...] = jnp.broadcast_to(res, (N_GRAPHS, D))


def kernel(x, edge_index, batch, edge_attr, W_rel1, b_rel1, W_root1,
           W_rel3, b_rel3, W_root3, W_lin, b_lin):
    f32 = jnp.float32
    pad = E_PAD - N_EDGES
    src2 = jnp.concatenate(
        [edge_index[0].astype(jnp.int32), jnp.zeros((pad,), jnp.int32)]
    ).reshape(EDGE_ROWS, CHUNK)
    dst2 = jnp.concatenate(
        [edge_index[1].astype(jnp.int32), jnp.zeros((pad,), jnp.int32)]
    ).reshape(EDGE_ROWS, CHUNK)
    w2 = jnp.concatenate(
        [edge_attr.astype(f32), jnp.zeros((pad,), f32)]
    ).reshape(EDGE_ROWS, CHUNK)
    zeros_nd = jnp.zeros((N_PAD, D), f32)

    y, z, u = pl.pallas_call(
        _pre_body,
        out_shape=(
            jax.ShapeDtypeStruct((N_NODES, D), f32),
            jax.ShapeDtypeStruct((N_NODES, D), f32),
            jax.ShapeDtypeStruct((8, D), f32),
        ),
    )(x, W_rel1, W_root1, b_rel1.reshape(1, D), W_rel3, W_root3, W_lin,
      b_rel3.reshape(1, D))

    mesh = plsc.VectorSubcoreMesh(core_axis_name="c", subcore_axis_name="s")
    sc_params = pltpu.CompilerParams(needs_layout_passes=False)
    agg = pl.kernel(
        _sc_conv_body,
        out_type=jax.ShapeDtypeStruct((NC, N_PAD, D), f32),
        mesh=mesh,
        compiler_params=sc_params,
        scratch_types=[
            pltpu.VMEM((CPT,), jnp.int32),
            pltpu.VMEM((CPT, CHUNK), jnp.int32),
            pltpu.VMEM((CPT, CHUNK), jnp.int32),
            pltpu.VMEM((CPT, CHUNK), f32),
            pltpu.VMEM((CHUNK, D), f32),
            pltpu.VMEM_SHARED((N_PAD, D), f32),
            pltpu.SemaphoreType.DMA,
        ],
    )(y, src2, dst2, w2, zeros_nd)

    scat = pl.pallas_call(
        _mid_body,
        grid=(5,),
        in_specs=[
            pl.BlockSpec((NC, N_NODES // 5, D), lambda i: (0, i, 0)),
            pl.BlockSpec((N_NODES // 5, D), lambda i: (i, 0)),
            pl.BlockSpec((8, D), lambda i: (0, 0)),
        ],
        out_specs=pl.BlockSpec((N_NODES // 5, 2), lambda i: (i, 0)),
        out_shape=jax.ShapeDtypeStruct((N_NODES, 2), f32),
    )(agg, z, u)

    s_rel = scat[:, 0]
    s_root = scat[:, 1].reshape(1, N_NODES)

    q = pl.kernel(
        _sc_q_body,
        out_type=jax.ShapeDtypeStruct((N_TILES, 1, N_NODES), f32),
        mesh=mesh,
        compiler_params=sc_params,
        scratch_types=[
            pltpu.VMEM((CPT,), jnp.int32),
            pltpu.VMEM((N_NODES,), f32),
            pltpu.VMEM((CPT, CHUNK), jnp.int32),
            pltpu.VMEM((CPT, CHUNK), jnp.int32),
            pltpu.VMEM((CPT, CHUNK), f32),
            pltpu.VMEM((1, N_NODES), f32),
            pltpu.SemaphoreType.DMA,
        ],
    )(s_rel, src2, dst2, w2)
    q = q.reshape(N_TILES, N_NODES)

    pooled = pl.pallas_call(
        _pool_body,
        out_shape=jax.ShapeDtypeStruct((N_GRAPHS, D), f32),
    )(q, s_root, batch.astype(jnp.int32).reshape(1, N_NODES), u,
      b_lin.reshape(1, 1))

    return pooled[:, :1]
